# baseline (speedup 1.0000x reference)
import jax
import jax.numpy as jnp
from jax import lax
from jax.experimental import pallas as pl
from jax.experimental.pallas import tpu as pltpu
from jax.experimental.pallas import tpu_sc as plsc

_NC = 2
_NS = 16
_NW = _NC * _NS

_N = 16384
_S = 26
_D = 64
_H = _N // 2
_JPW = _S // 2


def _bcast_body(emb_hbm, out_hbm, row_v, fb0, fb1, shared, sbuf, sem):
    core = lax.axis_index("c")
    sid = lax.axis_index("s")
    fbs = (fb0, fb1)

    pltpu.sync_copy(emb_hbm.at[pl.ds(0, _D)], row_v.at[pl.ds(0, _D)])

    handles = [None] * 8
    for t in range(4):
        c = sid * 4 + t
        bc = jnp.zeros((16,), jnp.float32) + row_v[pl.ds(c, 16)][0]
        fb = fbs[t % 2]
        if t >= 2:
            handles[2 * (t - 2)].wait()

        def fill(i, carry, fb=fb, bc=bc):
            for u in range(4):
                fb[pl.ds(i * 64 + u * 16, 16)] = bc
            return carry

        lax.fori_loop(0, _H // 64, fill, 0)
        handles[2 * t] = pltpu.async_copy(fb, shared.at[c], sem)
    for t in (2, 3):
        handles[2 * t].wait()
    plsc.subcore_barrier()

    ct = sid // 2
    j0 = (sid % 2) * _JPW
    pltpu.sync_copy(shared.at[pl.ds(ct * 8, 8)], sbuf)
    copies = [
        pltpu.async_copy(
            sbuf,
            out_hbm.at[pl.ds((j0 + n) * _D + ct * 8, 8), pl.ds(core * _H, _H)],
            sem,
        )
        for n in range(_JPW)
    ]
    for cp in copies:
        cp.wait()


def kernel(x, emb):
    mesh = plsc.VectorSubcoreMesh(
        core_axis_name="c", subcore_axis_name="s",
        num_cores=_NC, num_subcores=_NS,
    )
    run = pl.kernel(
        _bcast_body,
        out_type=jax.ShapeDtypeStruct((_S * _D, _N), jnp.float32),
        mesh=mesh,
        scratch_types=[
            pltpu.VMEM((_D + 16,), jnp.float32),
            pltpu.VMEM((_H,), jnp.float32),
            pltpu.VMEM((_H,), jnp.float32),
            pltpu.VMEM_SHARED((_D, _H), jnp.float32),
            pltpu.VMEM((8, _H), jnp.float32),
            pltpu.SemaphoreType.DMA,
        ],
    )
    out = run(emb.reshape(-1))
    return out.reshape(_S, _D, _N).transpose(2, 0, 1)


# speedup vs baseline: 1.0416x; 1.0416x over previous
import jax
import jax.numpy as jnp
from jax import lax
from jax.experimental import pallas as pl
from jax.experimental.pallas import tpu as pltpu
from jax.experimental.pallas import tpu_sc as plsc

_NC = 2
_NS = 16
_NW = _NC * _NS

_N = 16384
_S = 26
_D = 64
_CPW = _D // _NW


def _bcast_body(emb_hbm, out_hbm, row_v, buf_a, buf_b, sem):
    wid = lax.axis_index("s") * _NC + lax.axis_index("c")
    bufs = (buf_a, buf_b)

    pltpu.sync_copy(emb_hbm.at[pl.ds(0, _D)], row_v.at[pl.ds(0, _D)])

    copies = []
    for t in range(_CPW):
        c = wid * _CPW + t
        bc = jnp.zeros((16,), jnp.float32) + row_v[pl.ds(c, 16)][0]
        buf = bufs[t]

        def fill(i, carry, buf=buf, bc=bc):
            for u in range(4):
                buf[pl.ds(i * 64 + u * 16, 16)] = bc
            return carry

        lax.fori_loop(0, _N // 64, fill, 0)
        for j in range(_S):
            copies.append(
                pltpu.async_copy(bufs[t], out_hbm.at[j, c], sem)
            )
    for cp in copies:
        cp.wait()


def kernel(x, emb):
    mesh = plsc.VectorSubcoreMesh(
        core_axis_name="c", subcore_axis_name="s",
        num_cores=_NC, num_subcores=_NS,
    )
    run = pl.kernel(
        _bcast_body,
        out_type=jax.ShapeDtypeStruct((_S, _D, _N), jnp.float32),
        mesh=mesh,
        scratch_types=[
            pltpu.VMEM((_D + 16,), jnp.float32),
            pltpu.VMEM((_N,), jnp.float32),
            pltpu.VMEM((_N,), jnp.float32),
            pltpu.SemaphoreType.DMA,
        ],
    )
    out = run(emb.reshape(-1))
    return out.transpose(2, 0, 1)
